# baseline (speedup 1.0000x reference)
import jax
import jax.numpy as jnp
from jax.experimental import pallas as pl
from jax.experimental.pallas import tpu as pltpu

_NUM_CLASSES = 19
_IGNORE_INDEX = 255
_LANES = 128
_CBLK = 128


def _pool_kernel(gt_ref, f_ref, o_ref, wt_ref):
    j = pl.program_id(1)
    hw = wt_ref.shape[1]

    @pl.when(j == 0)
    def _():
        gt = gt_ref[0].reshape(1, hw)
        row = jax.lax.broadcasted_iota(jnp.int32, (_LANES, hw), 0)
        onehot = (gt == row).astype(jnp.float32)
        cnt = jnp.sum(onehot, axis=1, keepdims=True)
        wt_ref[...] = onehot / jnp.where(cnt > 0.0, cnt, 1.0)

    f = f_ref[0].reshape(f_ref.shape[1], hw)
    acc = jax.lax.dot_general(
        f, wt_ref[...],
        dimension_numbers=(((1,), (1,)), ((), ())),
        preferred_element_type=jnp.float32,
    )
    o_ref[0, :, :, 0] = acc[:, :_NUM_CLASSES]


def kernel(feats, gt_seg_map):
    B, C, H, W = feats.shape
    HW = H * W
    gt = gt_seg_map.astype(jnp.int32)

    out = pl.pallas_call(
        _pool_kernel,
        grid=(B, C // _CBLK),
        in_specs=[
            pl.BlockSpec((1, H, W), lambda b, j: (b, 0, 0)),
            pl.BlockSpec((1, _CBLK, H, W), lambda b, j: (b, j, 0, 0)),
        ],
        out_specs=pl.BlockSpec((1, _CBLK, _NUM_CLASSES, 1),
                               lambda b, j: (b, j, 0, 0)),
        out_shape=jax.ShapeDtypeStruct((B, C, _NUM_CLASSES, 1), jnp.float32),
        scratch_shapes=[
            pltpu.VMEM((_LANES, HW), jnp.float32),
        ],
        compiler_params=pltpu.CompilerParams(
            dimension_semantics=("parallel", "arbitrary"),
            vmem_limit_bytes=56 * 1024 * 1024,
        ),
        name="class_mean_pool",
    )(gt, feats)

    return out


# speedup vs baseline: 1.3168x; 1.3168x over previous
import jax
import jax.numpy as jnp
from jax.experimental import pallas as pl
from jax.experimental.pallas import tpu as pltpu

_NUM_CLASSES = 19
_IGNORE_INDEX = 255
_LANES = 128
_CBLK = 128


def _pool_kernel(gt_ref, f_ref, o_ref, wt_ref):
    j = pl.program_id(1)
    hw = wt_ref.shape[1]

    @pl.when(j == 0)
    def _():
        gt = gt_ref[0].reshape(1, hw)
        row = jax.lax.broadcasted_iota(jnp.int32, (_LANES, hw), 0)
        onehot = (gt == row).astype(jnp.float32)
        cnt = jnp.sum(onehot, axis=1, keepdims=True)
        wt_ref[...] = onehot / jnp.where(cnt > 0.0, cnt, 1.0)

    f = f_ref[0].reshape(f_ref.shape[1], hw)
    o_ref[0] = jax.lax.dot_general(
        f, wt_ref[...],
        dimension_numbers=(((1,), (1,)), ((), ())),
        preferred_element_type=jnp.float32,
    )


def kernel(feats, gt_seg_map):
    B, C, H, W = feats.shape
    HW = H * W
    gt = gt_seg_map.astype(jnp.int32)

    out = pl.pallas_call(
        _pool_kernel,
        grid=(B, C // _CBLK),
        in_specs=[
            pl.BlockSpec((1, H, W), lambda b, j: (b, 0, 0)),
            pl.BlockSpec((1, _CBLK, H, W), lambda b, j: (b, j, 0, 0)),
        ],
        out_specs=pl.BlockSpec((1, _CBLK, _LANES), lambda b, j: (b, j, 0)),
        out_shape=jax.ShapeDtypeStruct((B, C, _LANES), jnp.float32),
        scratch_shapes=[
            pltpu.VMEM((_LANES, HW), jnp.float32),
        ],
        compiler_params=pltpu.CompilerParams(
            dimension_semantics=("parallel", "arbitrary"),
            vmem_limit_bytes=56 * 1024 * 1024,
        ),
        name="class_mean_pool",
    )(gt, feats)

    return out[:, :, :_NUM_CLASSES, None]
